# Initial kernel scaffold; baseline (speedup 1.0000x reference)
import jax
import jax.numpy as jnp
from jax.experimental import pallas as pl


def kernel(l, ab, y, idx, memory_l, memory_ab):
    raise NotImplementedError("write your pallas kernel here")


# speedup vs baseline: 14.5228x; 14.5228x over previous
import functools

import jax
import jax.numpy as jnp
from jax import lax
from jax.experimental import pallas as pl
from jax.experimental.pallas import tpu as pltpu
from jax.experimental.pallas import tpu_sc as plsc

NC = 2
NS = 16
NW = NC * NS
L = 16

T = 0.07
MOMENTUM = 0.5
CHUNK = 128


def _sc_scores_body(mem_l, mem_ab, ab_h, l_h, idx_h, out_ab_h, out_l_h,
                    idx_v, vab_v, vl_v, rows0, rows1, oab_v, ol_v, sem0, sem1):
  kp1 = idx_h.shape[1] * idx_h.shape[2]
  n_chunks = kp1 // CHUNK
  b_per_w = ab_h.shape[0] // NW
  wid = lax.axis_index("s") * NC + lax.axis_index("c")
  lane = lax.iota(jnp.int32, L)

  perms = [lane ^ s for s in (8, 4, 2, 1)]

  def do_b(t, carry):
    b = wid * b_per_w + t
    pltpu.sync_copy(idx_h.at[b], idx_v)
    pltpu.sync_copy(ab_h.at[b], vab_v)
    pltpu.sync_copy(l_h.at[b], vl_v)
    va = [vab_v[pl.ds(q * L, L)] * (1.0 / T) for q in range(128 // L)]
    vb = [vl_v[pl.ds(q * L, L)] * (1.0 / T) for q in range(128 // L)]

    def fire(c):
      bank = mem_l if c < n_chunks else mem_ab
      buf = rows0 if c % 2 == 0 else rows1
      sem = sem0 if c % 2 == 0 else sem1
      return pltpu.async_copy(bank.at[idx_v.at[c % n_chunks]], buf, sem)

    total = 2 * n_chunks
    desc = fire(0)
    for c in range(total):
      nxt = fire(c + 1) if c + 1 < total else None
      desc.wait()
      buf = rows0 if c % 2 == 0 else rows1
      qv = va if c < n_chunks else vb
      out_v = oab_v if c < n_chunks else ol_v
      base = (c % n_chunks) * CHUNK
      for g in range(CHUNK // L):
        def kbody(i, vec, _g=g, _buf=buf, _qv=qv):
          k = _g * L + i
          acc = _buf[k, pl.ds(0, L)] * _qv[0]
          for q in range(1, 128 // L):
            acc = acc + _buf[k, pl.ds(q * L, L)] * _qv[q]
          for p in perms:
            acc = acc + jnp.take_along_axis(acc, p, axis=0)
          return jnp.where(lane == i, acc, vec)
        vec = lax.fori_loop(0, L, kbody, jnp.zeros((L,), jnp.float32))
        out_v[pl.ds(base + g * L, L)] = vec
      desc = nxt
    pltpu.sync_copy(oab_v, out_ab_h.at[b])
    pltpu.sync_copy(ol_v, out_l_h.at[b])
    return carry

  lax.fori_loop(0, b_per_w, do_b, 0)


def _sc_scores(memory_l, memory_ab, ab, l, idx3):
  b, kp1 = idx3.shape[0], idx3.shape[1] * idx3.shape[2]
  mesh = plsc.VectorSubcoreMesh(core_axis_name="c", subcore_axis_name="s")
  f = pl.kernel(
      _sc_scores_body,
      out_type=[jax.ShapeDtypeStruct((b, kp1), jnp.float32),
                jax.ShapeDtypeStruct((b, kp1), jnp.float32)],
      mesh=mesh,
      scratch_types=[
          pltpu.VMEM(idx3.shape[1:], jnp.int32),
          pltpu.VMEM((128,), jnp.float32),
          pltpu.VMEM((128,), jnp.float32),
          pltpu.VMEM((CHUNK, 128), jnp.float32),
          pltpu.VMEM((CHUNK, 128), jnp.float32),
          pltpu.VMEM((kp1,), jnp.float32),
          pltpu.VMEM((kp1,), jnp.float32),
          pltpu.SemaphoreType.DMA,
          pltpu.SemaphoreType.DMA,
      ],
  )
  return f(memory_l, memory_ab, ab, l, idx3)


def _tc_update_body(y_ref, mlrow, marow, lrow, abrow, ml_any, mab_any,
                    oml, omab):
  del y_ref, ml_any, mab_any
  vl = mlrow[...] * MOMENTUM + lrow[...] * (1.0 - MOMENTUM)
  oml[...] = vl / jnp.sqrt(jnp.sum(vl * vl, axis=-1, keepdims=True))
  vab = marow[...] * MOMENTUM + abrow[...] * (1.0 - MOMENTUM)
  omab[...] = vab / jnp.sqrt(jnp.sum(vab * vab, axis=-1, keepdims=True))


def _tc_update(y, memory_l, memory_ab, l, ab):
  b = y.shape[0]
  n, d = memory_l.shape
  ml3 = memory_l.reshape(n, 1, d)
  mab3 = memory_ab.reshape(n, 1, d)
  l3 = l.reshape(b, 1, d)
  ab3 = ab.reshape(b, 1, d)
  row = pl.BlockSpec((1, 1, d), lambda i, y_ref: (y_ref[i], 0, 0))
  batch_row = pl.BlockSpec((1, 1, d), lambda i, y_ref: (i, 0, 0))
  grid_spec = pltpu.PrefetchScalarGridSpec(
      num_scalar_prefetch=1,
      grid=(b,),
      in_specs=[row, row, batch_row, batch_row,
                pl.BlockSpec(memory_space=pl.ANY),
                pl.BlockSpec(memory_space=pl.ANY)],
      out_specs=[pl.BlockSpec((1, 1, d), lambda i, y_ref: (y_ref[i], 0, 0)),
                 pl.BlockSpec((1, 1, d), lambda i, y_ref: (y_ref[i], 0, 0))],
  )
  oml, omab = pl.pallas_call(
      _tc_update_body,
      grid_spec=grid_spec,
      out_shape=[jax.ShapeDtypeStruct((n, 1, d), jnp.float32),
                 jax.ShapeDtypeStruct((n, 1, d), jnp.float32)],
      input_output_aliases={5: 0, 6: 1},
  )(y, ml3, mab3, l3, ab3, ml3, mab3)
  return oml.reshape(n, d), omab.reshape(n, d)


def kernel(l, ab, y, idx, memory_l, memory_ab):
  b, kp1 = idx.shape
  idx3 = idx.astype(jnp.int32).reshape(b, kp1 // CHUNK, CHUNK)
  out_ab, out_l = _sc_scores(memory_l, memory_ab, ab, l, idx3)
  new_ml, new_mab = _tc_update(y.astype(jnp.int32), memory_l, memory_ab, l, ab)
  return (out_l.reshape(b, kp1, 1), out_ab.reshape(b, kp1, 1),
          new_ml, new_mab)


# speedup vs baseline: 22.4099x; 1.5431x over previous
import jax
import jax.numpy as jnp
from jax import lax
from jax.experimental import pallas as pl
from jax.experimental.pallas import tpu as pltpu
from jax.experimental.pallas import tpu_sc as plsc

NC = 2
NS = 16
NW = NC * NS
L = 16

T = 0.07
MOMENTUM = 0.5
CHUNK = 128


NBUF = 4


def _sc_scores_body(mem_l, mem_ab, ab_h, l_h, idx_h, out_ab_h, out_l_h,
                    idx_all, vab_all, vl_all, bufs, out_all, sems):
  n_chunks = idx_h.shape[1]
  b_per_w = ab_h.shape[0] // NW
  assert n_chunks == NBUF
  wid = lax.axis_index("s") * NC + lax.axis_index("c")
  lane = lax.iota(jnp.int32, L)

  pltpu.sync_copy(idx_h.at[pl.ds(wid * b_per_w, b_per_w)], idx_all)
  pltpu.sync_copy(ab_h.at[pl.ds(wid * b_per_w, b_per_w)], vab_all)
  pltpu.sync_copy(l_h.at[pl.ds(wid * b_per_w, b_per_w)], vl_all)

  def merge(a, b, s):
    pred = (lane & s) == 0
    u = jnp.where(pred, a, b)
    v = jnp.where(pred, b, a)
    return u + jnp.take_along_axis(v, lane ^ s, axis=0)

  def run_bank(bank, vec_all, out_h):
    def fire(b_local, r):
      return pltpu.async_copy(bank.at[idx_all.at[b_local, r]],
                              bufs.at[r], sems.at[r])

    def drain(b_local, r):
      pltpu.make_async_copy(bank.at[idx_all.at[b_local, r]],
                            bufs.at[r], sems.at[r]).wait()

    for r in range(NBUF):
      fire(0, r)

    def do_b(g, carry):
      qv = [vec_all[g, pl.ds(q * L, L)] * (1.0 / T) for q in range(128 // L)]
      for r in range(NBUF):
        drain(g, r)

        def gbody(gg, c, _r=r, _qv=qv, _g=g):
          accs = []
          for j in range(L):
            k = gg * L + j
            acc = bufs[_r, k, pl.ds(0, L)] * _qv[0]
            for q in range(1, 128 // L):
              acc = acc + bufs[_r, k, pl.ds(q * L, L)] * _qv[q]
            accs.append(acc)
          for s in (1, 2, 4, 8):
            accs = [merge(accs[2 * m], accs[2 * m + 1], s)
                    for m in range(len(accs) // 2)]
          out_all[_g, pl.ds(_r * CHUNK + gg * L, L)] = accs[0]
          return c

        lax.fori_loop(0, CHUNK // L, gbody, 0)

        @pl.when(g + 1 < b_per_w)
        def _():
          fire(g + 1, r)
      return carry

    lax.fori_loop(0, b_per_w, do_b, 0)
    pltpu.sync_copy(out_all, out_h.at[pl.ds(wid * b_per_w, b_per_w)])

  run_bank(mem_l, vab_all, out_ab_h)
  run_bank(mem_ab, vl_all, out_l_h)


def _sc_scores(memory_l, memory_ab, ab, l, idx3):
  b, kp1 = idx3.shape[0], idx3.shape[1] * idx3.shape[2]
  mesh = plsc.VectorSubcoreMesh(core_axis_name="c", subcore_axis_name="s")
  f = pl.kernel(
      _sc_scores_body,
      out_type=[jax.ShapeDtypeStruct((b, kp1), jnp.float32),
                jax.ShapeDtypeStruct((b, kp1), jnp.float32)],
      mesh=mesh,
      scratch_types=[
          pltpu.VMEM((b // NW,) + idx3.shape[1:], jnp.int32),
          pltpu.VMEM((b // NW, 128), jnp.float32),
          pltpu.VMEM((b // NW, 128), jnp.float32),
          pltpu.VMEM((NBUF, CHUNK, 128), jnp.float32),
          pltpu.VMEM((b // NW, kp1), jnp.float32),
          pltpu.SemaphoreType.DMA((NBUF,)),
      ],
  )
  return f(memory_l, memory_ab, ab, l, idx3)


COPY_BLOCK = 10000


def _copy_body(ml_ref, mab_ref, ol_ref, oab_ref):
  ol_ref[...] = ml_ref[...]
  oab_ref[...] = mab_ref[...]


def _tc_copy(memory_l, memory_ab):
  n, d = memory_l.shape
  blk = pl.BlockSpec((COPY_BLOCK, d), lambda i: (i, 0))
  return pl.pallas_call(
      _copy_body,
      grid=(n // COPY_BLOCK,),
      in_specs=[blk, blk],
      out_specs=[blk, blk],
      out_shape=[jax.ShapeDtypeStruct((n, d), jnp.float32),
                 jax.ShapeDtypeStruct((n, d), jnp.float32)],
  )(memory_l, memory_ab)


def _tc_update_body(y_ref, l_ref, ab_ref, ml_any, mab_any, oml, omab,
                    rows_l, rows_ab, upd_l, upd_ab, sem_rl, sem_rab,
                    sem_wl, sem_wab):
  b = l_ref.shape[0]

  def rd(i, carry):
    yb = y_ref[i]
    pltpu.async_copy(ml_any.at[yb], rows_l.at[i], sem_rl)
    pltpu.async_copy(mab_any.at[yb], rows_ab.at[i], sem_rab)
    return carry
  lax.fori_loop(0, b, rd, 0)
  pltpu.make_async_copy(ml_any.at[pl.ds(0, b)], rows_l, sem_rl).wait()
  pltpu.make_async_copy(mab_any.at[pl.ds(0, b)], rows_ab, sem_rab).wait()

  vl = rows_l[...] * MOMENTUM + l_ref[...] * (1.0 - MOMENTUM)
  upd_l[...] = vl / jnp.sqrt(jnp.sum(vl * vl, axis=-1, keepdims=True))
  vab = rows_ab[...] * MOMENTUM + ab_ref[...] * (1.0 - MOMENTUM)
  upd_ab[...] = vab / jnp.sqrt(jnp.sum(vab * vab, axis=-1, keepdims=True))

  def wr(i, carry):
    yb = y_ref[i]
    pltpu.async_copy(upd_l.at[i], oml.at[yb], sem_wl)
    pltpu.async_copy(upd_ab.at[i], omab.at[yb], sem_wab)
    return carry
  lax.fori_loop(0, b, wr, 0)
  pltpu.make_async_copy(upd_l, oml.at[pl.ds(0, b)], sem_wl).wait()
  pltpu.make_async_copy(upd_ab, omab.at[pl.ds(0, b)], sem_wab).wait()


def _tc_update(y, memory_l, memory_ab, l, ab):
  b = y.shape[0]
  n, d = memory_l.shape
  return pl.pallas_call(
      _tc_update_body,
      in_specs=[pl.BlockSpec(memory_space=pltpu.SMEM),
                pl.BlockSpec((b, d), lambda: (0, 0)),
                pl.BlockSpec((b, d), lambda: (0, 0)),
                pl.BlockSpec(memory_space=pl.ANY),
                pl.BlockSpec(memory_space=pl.ANY)],
      out_specs=[pl.BlockSpec(memory_space=pl.ANY),
                 pl.BlockSpec(memory_space=pl.ANY)],
      out_shape=[jax.ShapeDtypeStruct((n, d), jnp.float32),
                 jax.ShapeDtypeStruct((n, d), jnp.float32)],
      scratch_shapes=[pltpu.VMEM((b, d), jnp.float32),
                      pltpu.VMEM((b, d), jnp.float32),
                      pltpu.VMEM((b, d), jnp.float32),
                      pltpu.VMEM((b, d), jnp.float32),
                      pltpu.SemaphoreType.DMA,
                      pltpu.SemaphoreType.DMA,
                      pltpu.SemaphoreType.DMA,
                      pltpu.SemaphoreType.DMA],
      input_output_aliases={3: 0, 4: 1},
  )(y, l, ab, memory_l, memory_ab)


def kernel(l, ab, y, idx, memory_l, memory_ab):
  b, kp1 = idx.shape
  idx3 = idx.astype(jnp.int32).reshape(b, kp1 // CHUNK, CHUNK)
  cml, cmab = _tc_copy(memory_l, memory_ab)
  new_ml, new_mab = _tc_update(y.astype(jnp.int32), cml, cmab, l, ab)
  out_ab, out_l = _sc_scores(memory_l, memory_ab, ab, l, idx3)
  return (out_l.reshape(b, kp1, 1), out_ab.reshape(b, kp1, 1),
          new_ml, new_mab)
